# baseline (speedup 1.0000x reference)
import functools

import jax
import jax.numpy as jnp
from jax import lax
from jax.experimental import pallas as pl
from jax.experimental.pallas import tpu as pltpu
from jax.experimental.pallas import tpu_sc as plsc

BATCH = 16384
EMBED_DIM = 32
TABLE_ROWS = 1000000
NUM_CHUNKS = 4
CHUNK = 128
PER_WORKER = NUM_CHUNKS * CHUNK


def _sc_body(uw_hbm, iw_hbm, utab_hbm, itab_hbm, out_hbm,
             uw_i, iw_i, u_v, v_v, out_v, sem_u, sem_v):
    nc = 2
    wid = lax.axis_index("s") * nc + lax.axis_index("c")

    pltpu.sync_copy(uw_hbm.at[wid], uw_i)
    pltpu.sync_copy(iw_hbm.at[wid], iw_i)

    def fire(d, _):
        for j in range(NUM_CHUNKS):
            pltpu.async_copy(utab_hbm.at[uw_i.at[d, j]], u_v.at[d, j], sem_u)
            pltpu.async_copy(itab_hbm.at[iw_i.at[d, j]], v_v.at[d, j], sem_v)
        return 0

    lax.fori_loop(0, EMBED_DIM, fire, 0)

    def drain(d, _):
        for j in range(NUM_CHUNKS):
            pltpu.make_async_copy(
                utab_hbm.at[uw_i.at[d, j]], u_v.at[d, j], sem_u).wait()
            pltpu.make_async_copy(
                itab_hbm.at[iw_i.at[d, j]], v_v.at[d, j], sem_v).wait()
        return 0

    lax.fori_loop(0, EMBED_DIM, drain, 0)

    def group_body(g, _):
        j = g // 8
        off = (g % 8) * 16
        acc = jnp.zeros((16,), jnp.float32)
        for d in range(EMBED_DIM):
            acc = acc + u_v[d, j, pl.ds(off, 16)] * v_v[d, j, pl.ds(off, 16)]
        out_v[j, pl.ds(off, 16)] = acc
        return 0

    lax.fori_loop(0, PER_WORKER // 16, group_body, 0)

    pltpu.sync_copy(out_v, out_hbm.at[wid])


@jax.jit
def _mf_dot(uw, iw, utab_flat, itab_flat):
    mesh = plsc.VectorSubcoreMesh(core_axis_name="c", subcore_axis_name="s")
    f = functools.partial(
        pl.kernel,
        mesh=mesh,
        compiler_params=pltpu.CompilerParams(use_tc_tiling_on_sc=False),
        out_type=jax.ShapeDtypeStruct((32, NUM_CHUNKS, CHUNK), jnp.float32),
        scratch_types=[
            pltpu.VMEM((EMBED_DIM, NUM_CHUNKS, CHUNK), jnp.int32),
            pltpu.VMEM((EMBED_DIM, NUM_CHUNKS, CHUNK), jnp.int32),
            pltpu.VMEM((EMBED_DIM, NUM_CHUNKS, CHUNK), jnp.float32),
            pltpu.VMEM((EMBED_DIM, NUM_CHUNKS, CHUNK), jnp.float32),
            pltpu.VMEM((NUM_CHUNKS, CHUNK), jnp.float32),
            pltpu.SemaphoreType.DMA,
            pltpu.SemaphoreType.DMA,
        ],
    )(_sc_body)
    return f(uw, iw, utab_flat, itab_flat)


def _word_indices(ids):
    w = ids.reshape(32, 1, PER_WORKER)
    d = (jnp.arange(EMBED_DIM, dtype=jnp.int32) * TABLE_ROWS).reshape(1, -1, 1)
    return (w + d).reshape(32, EMBED_DIM, NUM_CHUNKS, CHUNK)


def kernel(X, user_table, item_table):
    uw = _word_indices(X[:, 0])
    iw = _word_indices(X[:, 1])
    utab_flat = user_table.T.reshape(-1)
    itab_flat = item_table.T.reshape(-1)
    out = _mf_dot(uw, iw, utab_flat, itab_flat)
    return out.reshape(BATCH, 1)


# speedup vs baseline: 1.0017x; 1.0017x over previous
import functools

import jax
import jax.numpy as jnp
from jax import lax
from jax.experimental import pallas as pl
from jax.experimental.pallas import tpu as pltpu
from jax.experimental.pallas import tpu_sc as plsc

BATCH = 16384
EMBED_DIM = 32
TABLE_ROWS = 1000000
NUM_CHUNKS = 4
CHUNK = 128
PER_WORKER = NUM_CHUNKS * CHUNK
WORDS = EMBED_DIM * PER_WORKER


def _sc_body(uw_hbm, iw_hbm, utab_hbm, itab_hbm, out_hbm,
             uw_i, iw_i, u_v, v_v, out_v, drain_v, sem_u, sem_v):
    nc = 2
    wid = lax.axis_index("s") * nc + lax.axis_index("c")

    pltpu.sync_copy(uw_hbm.at[wid], uw_i)
    pltpu.sync_copy(iw_hbm.at[wid], iw_i)

    def fire(d, _):
        for j in range(NUM_CHUNKS):
            for k in range(CHUNK // 16):
                s = pl.ds(k * 16, 16)
                uidx = uw_i[d, j, s]
                iidx = iw_i[d, j, s]
                pltpu.async_copy(utab_hbm.at[uidx], u_v.at[d, j, s], sem_u)
                pltpu.async_copy(itab_hbm.at[iidx], v_v.at[d, j, s], sem_v)
        return 0

    lax.fori_loop(0, EMBED_DIM, fire, 0)

    pltpu.make_async_copy(
        utab_hbm.at[pl.ds(0, WORDS)], drain_v, sem_u).wait()
    pltpu.make_async_copy(
        itab_hbm.at[pl.ds(0, WORDS)], drain_v, sem_v).wait()

    def group_body(g, _):
        j = g // 8
        off = (g % 8) * 16
        acc = jnp.zeros((16,), jnp.float32)
        for d in range(EMBED_DIM):
            acc = acc + u_v[d, j, pl.ds(off, 16)] * v_v[d, j, pl.ds(off, 16)]
        out_v[j, pl.ds(off, 16)] = acc
        return 0

    lax.fori_loop(0, PER_WORKER // 16, group_body, 0)

    pltpu.sync_copy(out_v, out_hbm.at[wid])


@jax.jit
def _mf_dot(uw, iw, utab_flat, itab_flat):
    mesh = plsc.VectorSubcoreMesh(core_axis_name="c", subcore_axis_name="s")
    f = functools.partial(
        pl.kernel,
        mesh=mesh,
        compiler_params=pltpu.CompilerParams(use_tc_tiling_on_sc=False),
        out_type=jax.ShapeDtypeStruct((32, NUM_CHUNKS, CHUNK), jnp.float32),
        scratch_types=[
            pltpu.VMEM((EMBED_DIM, NUM_CHUNKS, CHUNK), jnp.int32),
            pltpu.VMEM((EMBED_DIM, NUM_CHUNKS, CHUNK), jnp.int32),
            pltpu.VMEM((EMBED_DIM, NUM_CHUNKS, CHUNK), jnp.float32),
            pltpu.VMEM((EMBED_DIM, NUM_CHUNKS, CHUNK), jnp.float32),
            pltpu.VMEM((NUM_CHUNKS, CHUNK), jnp.float32),
            pltpu.VMEM((WORDS,), jnp.float32),
            pltpu.SemaphoreType.DMA,
            pltpu.SemaphoreType.DMA,
        ],
    )(_sc_body)
    return f(uw, iw, utab_flat, itab_flat)


def _word_indices(ids):
    w = ids.reshape(32, 1, PER_WORKER)
    d = (jnp.arange(EMBED_DIM, dtype=jnp.int32) * TABLE_ROWS).reshape(1, -1, 1)
    return (w + d).reshape(32, EMBED_DIM, NUM_CHUNKS, CHUNK)


def kernel(X, user_table, item_table):
    uw = _word_indices(X[:, 0])
    iw = _word_indices(X[:, 1])
    utab_flat = user_table.T.reshape(-1)
    itab_flat = item_table.T.reshape(-1)
    out = _mf_dot(uw, iw, utab_flat, itab_flat)
    return out.reshape(BATCH, 1)


# speedup vs baseline: 1.0041x; 1.0024x over previous
import functools

import jax
import jax.numpy as jnp
from jax import lax
from jax.experimental import pallas as pl
from jax.experimental.pallas import tpu as pltpu
from jax.experimental.pallas import tpu_sc as plsc

BATCH = 16384
EMBED_DIM = 32
TABLE_ROWS = 1000000
PER_WORKER = 512
DPC = 2
ROUNDS = EMBED_DIM // DPC
ROUND_LINES = DPC * PER_WORKER


def _sc_body(uwa_hbm, iwa_hbm, uoff_hbm, ioff_hbm, utab_hbm, itab_hbm,
             out_hbm, uwa_i, iwa_i, uoff_i, ioff_i, stage_u, stage_v,
             out_v, drain_v, sem_u0, sem_u1, sem_v0, sem_v1):
    nc = 2
    wid = lax.axis_index("s") * nc + lax.axis_index("c")

    pltpu.sync_copy(uwa_hbm.at[wid], uwa_i)
    pltpu.sync_copy(iwa_hbm.at[wid], iwa_i)
    pltpu.sync_copy(uoff_hbm.at[wid], uoff_i)
    pltpu.sync_copy(ioff_hbm.at[wid], ioff_i)

    def zero_body(g, _):
        out_v[pl.ds(g * 16, 16)] = jnp.zeros((16,), jnp.float32)
        return 0

    lax.fori_loop(0, PER_WORKER // 16, zero_body, 0)

    def fire(k, su, sv):
        for dd in range(DPC):
            for c in range(PER_WORKER // 128):
                s = pl.ds(c * 128, 128)
                pltpu.async_copy(
                    utab_hbm.at[uwa_i.at[k * DPC + dd, s]],
                    stage_u.at[lax.rem(k, 2), dd, s], su)
                pltpu.async_copy(
                    itab_hbm.at[iwa_i.at[k * DPC + dd, s]],
                    stage_v.at[lax.rem(k, 2), dd, s], sv)

    def drain(su, sv):
        pltpu.make_async_copy(
            utab_hbm.at[pl.ds(0, ROUND_LINES)], drain_v, su).wait()
        pltpu.make_async_copy(
            itab_hbm.at[pl.ds(0, ROUND_LINES)], drain_v, sv).wait()

    lanes = lax.iota(jnp.int32, 16)

    def _bcast(vec, i):
        return vec.at[jnp.full((16,), i, jnp.int32)].get(
            mode="promise_in_bounds")

    def compute(k, _):
        buf = lax.rem(k, 2)

        def group_body(g, _):
            s = pl.ds(g * 16, 16)
            uoff = uoff_i[s]
            ioff = ioff_i[s]
            acc = out_v[s]
            for i in range(16):
                row = g * 16 + i
                obu = _bcast(uoff, i)
                obv = _bcast(ioff, i)
                w = jnp.zeros((16,), jnp.float32)
                for dd in range(DPC):
                    lu = stage_u[buf, dd, row, pl.ds(0, 16)]
                    lv = stage_v[buf, dd, row, pl.ds(0, 16)]
                    uw = lu.at[obu].get(mode="promise_in_bounds")
                    vw = lv.at[obv].get(mode="promise_in_bounds")
                    w = w + uw * vw
                acc = jnp.where(lanes == i, acc + w, acc)
            out_v[s] = acc
            return 0

        lax.fori_loop(0, PER_WORKER // 16, group_body, 0)
        return 0

    fire(0, sem_u0, sem_v0)

    def round_body(k, _):
        par = lax.rem(k, 2)

        @pl.when(k + 1 < ROUNDS)
        def _():
            @pl.when(par == 0)
            def _():
                fire(k + 1, sem_u1, sem_v1)

            @pl.when(par == 1)
            def _():
                fire(k + 1, sem_u0, sem_v0)

        @pl.when(par == 0)
        def _():
            drain(sem_u0, sem_v0)

        @pl.when(par == 1)
        def _():
            drain(sem_u1, sem_v1)

        compute(k, 0)
        return 0

    lax.fori_loop(0, ROUNDS, round_body, 0)

    pltpu.sync_copy(out_v, out_hbm.at[wid])


@jax.jit
def _mf_dot(uwa, iwa, uoff, ioff, utab_flat, itab_flat):
    mesh = plsc.VectorSubcoreMesh(core_axis_name="c", subcore_axis_name="s")
    f = functools.partial(
        pl.kernel,
        mesh=mesh,
        compiler_params=pltpu.CompilerParams(use_tc_tiling_on_sc=False),
        out_type=jax.ShapeDtypeStruct((32, PER_WORKER), jnp.float32),
        scratch_types=[
            pltpu.VMEM((EMBED_DIM, PER_WORKER), jnp.int32),
            pltpu.VMEM((EMBED_DIM, PER_WORKER), jnp.int32),
            pltpu.VMEM((PER_WORKER,), jnp.int32),
            pltpu.VMEM((PER_WORKER,), jnp.int32),
            pltpu.VMEM((2, DPC, PER_WORKER, 16), jnp.float32),
            pltpu.VMEM((2, DPC, PER_WORKER, 16), jnp.float32),
            pltpu.VMEM((PER_WORKER,), jnp.float32),
            pltpu.VMEM((ROUND_LINES, 16), jnp.float32),
            pltpu.SemaphoreType.DMA,
            pltpu.SemaphoreType.DMA,
            pltpu.SemaphoreType.DMA,
            pltpu.SemaphoreType.DMA,
        ],
    )(_sc_body)
    return f(uwa, iwa, uoff, ioff, utab_flat, itab_flat)


def _aligned_indices(ids):
    w = (ids >> 4).reshape(32, 1, PER_WORKER)
    d = (jnp.arange(EMBED_DIM, dtype=jnp.int32) *
         (TABLE_ROWS // 16)).reshape(1, -1, 1)
    return w + d


def kernel(X, user_table, item_table):
    uid = X[:, 0]
    iid = X[:, 1]
    uwa = _aligned_indices(uid)
    iwa = _aligned_indices(iid)
    uoff = (uid & 15).reshape(32, PER_WORKER)
    ioff = (iid & 15).reshape(32, PER_WORKER)
    utab_flat = user_table.T.reshape(-1, 16)
    itab_flat = item_table.T.reshape(-1, 16)
    out = _mf_dot(uwa, iwa, uoff, ioff, utab_flat, itab_flat)
    return out.reshape(BATCH, 1)
